# Initial kernel scaffold; baseline (speedup 1.0000x reference)
import jax
import jax.numpy as jnp
from jax.experimental import pallas as pl


def kernel(x, edge_index, edge_weight, pos_edge_index, W1, W2, W_lin):
    raise NotImplementedError("write your pallas kernel here")


# speedup vs baseline: 4.7957x; 4.7957x over previous
import functools

import jax
import jax.numpy as jnp
from jax import lax
from jax.experimental import pallas as pl
from jax.experimental.pallas import tpu as pltpu
from jax.experimental.pallas import tpu_sc as plsc

N = 10000
N_PAD = 10240
E = 320000
P = 20000
F = 128

NC = 2
NS = 16
NW = NC * NS
CHUNK = 128
NCHUNKS = E // CHUNK
ROWS_PER_TILE = N_PAD // NS
LANES = 16

P_PAD = 20480
POS_PER_W = P_PAD // NW


def _mesh():
    return plsc.VectorSubcoreMesh(core_axis_name="c", subcore_axis_name="s")


def _mm_body(x_ref, w_ref, o_ref):
    o_ref[...] = jnp.dot(x_ref[...], w_ref[...],
                         preferred_element_type=jnp.float32)


def _mm_partials_body(p_ref, w_ref, o_ref, *, relu):
    h = p_ref[0] + p_ref[1]
    if relu:
        h = jnp.maximum(h, 0.0)
    o_ref[...] = jnp.dot(h, w_ref[...], preferred_element_type=jnp.float32)


def _tc_mm(x, w):
    return pl.pallas_call(
        _mm_body,
        out_shape=jax.ShapeDtypeStruct((x.shape[0], w.shape[1]), jnp.float32),
    )(x, w)


def _tc_mm_partials(p, w, relu):
    return pl.pallas_call(
        functools.partial(_mm_partials_body, relu=relu),
        out_shape=jax.ShapeDtypeStruct((p.shape[1], w.shape[1]), jnp.float32),
    )(p, w)


def _gcn_aggregate(h, src, dst, w):
    @functools.partial(
        pl.kernel,
        mesh=_mesh(),
        out_type=jax.ShapeDtypeStruct((NC, N_PAD, F), jnp.float32),
        scratch_types=[
            pltpu.VMEM((CHUNK,), jnp.int32),
            pltpu.VMEM((CHUNK,), jnp.int32),
            pltpu.VMEM((CHUNK,), jnp.float32),
            pltpu.VMEM((CHUNK, F), jnp.float32),
            pltpu.VMEM_SHARED((N_PAD, F), jnp.float32),
            pltpu.SemaphoreType.DMA,
        ],
    )
    def agg(h_hbm, src_hbm, dst_hbm, w_hbm, out_hbm,
            sidx_v, didx_v, w_v, rows_v, acc, gsem):
        cid = lax.axis_index("c")
        sid = lax.axis_index("s")
        wid = sid * NC + cid

        def zrow(i, _):
            for j in range(F // LANES):
                rows_v[i, pl.ds(LANES * j, LANES)] = jnp.zeros(
                    (LANES,), jnp.float32)
            return 0
        lax.fori_loop(0, CHUNK, zrow, 0)
        for m in range(ROWS_PER_TILE // CHUNK):
            pltpu.sync_copy(
                rows_v,
                acc.at[pl.ds(sid * ROWS_PER_TILE + CHUNK * m, CHUNK)])
        plsc.subcore_barrier()

        nchunks = NCHUNKS // NW + jnp.where(wid < NCHUNKS % NW, 1, 0)

        def chunk_body(k, _):
            base = (wid + NW * k) * CHUNK
            pltpu.sync_copy(src_hbm.at[pl.ds(base, CHUNK)], sidx_v)
            pltpu.sync_copy(dst_hbm.at[pl.ds(base, CHUNK)], didx_v)
            pltpu.sync_copy(w_hbm.at[pl.ds(base, CHUNK)], w_v)
            pltpu.async_copy(h_hbm.at[sidx_v], rows_v, gsem).wait()

            def mul_group(g, _):
                w16 = w_v[pl.ds(g * LANES, LANES)]
                for l in range(LANES):
                    wl = w16[l]
                    i = g * LANES + l
                    for j in range(F // LANES):
                        sl = pl.ds(LANES * j, LANES)
                        rows_v[i, sl] = rows_v[i, sl] * wl
                return 0
            lax.fori_loop(0, CHUNK // LANES, mul_group, 0)

            pltpu.sync_copy(rows_v, acc.at[didx_v], add=True)
            return 0
        lax.fori_loop(0, nchunks, chunk_body, 0)
        plsc.subcore_barrier()

        r0 = sid * ROWS_PER_TILE
        pltpu.sync_copy(acc.at[pl.ds(r0, ROWS_PER_TILE)],
                        out_hbm.at[cid, pl.ds(r0, ROWS_PER_TILE)])

    return agg(h, src, dst, w)


POS_CHUNKS_PER_W = POS_PER_W // CHUNK


def _decode(zab, pos0, pos1):
    @functools.partial(
        pl.kernel,
        mesh=_mesh(),
        out_type=jax.ShapeDtypeStruct((P_PAD, LANES), jnp.float32),
        scratch_types=[
            pltpu.VMEM((CHUNK,), jnp.int32),
            pltpu.VMEM((CHUNK,), jnp.int32),
            pltpu.VMEM((CHUNK, F), jnp.float32),
            pltpu.VMEM((CHUNK, F), jnp.float32),
            pltpu.VMEM((CHUNK, LANES), jnp.float32),
            pltpu.SemaphoreType.DMA,
            pltpu.SemaphoreType.DMA,
        ],
    )
    def dec(zab_hbm, p0_hbm, p1_hbm, out_hbm,
            p0_v, p1_v, ra_v, rb_v, o_v, sema, semb):
        cid = lax.axis_index("c")
        sid = lax.axis_index("s")
        wid = sid * NC + cid

        def chunk_body(k, _):
            base = wid * POS_PER_W + k * CHUNK
            pltpu.sync_copy(p0_hbm.at[pl.ds(base, CHUNK)], p0_v)
            pltpu.sync_copy(p1_hbm.at[pl.ds(base, CHUNK)], p1_v)
            cpa = pltpu.async_copy(zab_hbm.at[p0_v], ra_v, sema)
            cpb = pltpu.async_copy(zab_hbm.at[p1_v], rb_v, semb)
            cpa.wait()
            cpb.wait()

            def add_body(i, _):
                o_v[i, pl.ds(0, LANES)] = (ra_v[i, pl.ds(0, LANES)]
                                           + rb_v[i, pl.ds(LANES, LANES)])
                return 0
            lax.fori_loop(0, CHUNK, add_body, 0)
            pltpu.sync_copy(o_v, out_hbm.at[pl.ds(base, CHUNK)])
            return 0
        lax.fori_loop(0, POS_CHUNKS_PER_W, chunk_body, 0)

    return dec(zab, pos0, pos1)


def kernel(x, edge_index, edge_weight, pos_edge_index, W1, W2, W_lin):
    x = x.astype(jnp.float32)
    src = edge_index[0]
    dst = edge_index[1]

    h1 = _tc_mm(x, W1)
    p1 = _gcn_aggregate(h1, src, dst, edge_weight)
    h2 = _tc_mm_partials(p1, W2, relu=True)
    p2 = _gcn_aggregate(h2, src, dst, edge_weight)

    wcat = jnp.zeros((F, F), jnp.float32)
    wcat = wcat.at[:, 0:2].set(W_lin[:F])
    wcat = wcat.at[:, LANES:LANES + 2].set(W_lin[F:])
    zab = _tc_mm_partials(p2, wcat, relu=False)

    pos = jnp.pad(pos_edge_index, ((0, 0), (0, P_PAD - P)))
    o2 = _decode(zab, pos[0], pos[1])
    return o2[:P, :2]
